# baseline (speedup 1.0000x reference)
import functools

import jax
import jax.numpy as jnp
from jax import lax
from jax.experimental import pallas as pl
from jax.experimental.pallas import tpu as pltpu
from jax.experimental.pallas import tpu_sc as plsc

Q = 1024
D = 768
K = 100000
TOPK = 100
TK = 2048
GRID = (K + TK - 1) // TK
KP = GRID * TK

NB = 2048
CAP = 1024
WCH = 1024
NCH = KP // WCH
SB = 64
SUBCH = 128
NSUB = KP // SUBCH
NSUBP = 896
NC = 2
NS = 16
NW = NC * NS
RPW = Q // NW
NGR = RPW // 16

_I32MIN = -0x80000000


def _pooler_body(qh_ref, w_ref, b_ref, e_ref):
    acc = jax.lax.dot_general(
        qh_ref[...], w_ref[...], (((1,), (0,)), ((), ())),
        preferred_element_type=jnp.float32,
    )
    e_ref[...] = jnp.tanh(acc + b_ref[...])


def _scores_body(e_ref, p_ref, s_ref, sm_ref):
    i = pl.program_id(0)
    acc = jax.lax.dot_general(
        e_ref[...], p_ref[...], (((1,), (1,)), ((), ())),
        preferred_element_type=jnp.float32,
    )
    col = i * TK + jax.lax.broadcasted_iota(jnp.int32, (Q, TK), 1)
    masked = jnp.where(col < K, acc, jnp.finfo(jnp.float32).min)
    s_ref[...] = masked
    sm_ref[...] = jnp.max(masked.reshape(Q, TK // SUBCH, SUBCH),
                          axis=2)[None]


def _topk_sc_body(scores, submax, vals_out, idx_out,
                  chunk, hist, ck, ci, ck2, ci2, rhist, outv, outi, sem):
    wid = lax.axis_index("s") * NC + lax.axis_index("c")
    lane = lax.iota(jnp.int32, 16)
    ones = jnp.ones((16,), jnp.int32)
    zeros = jnp.zeros((16,), jnp.int32)
    lane_chunk = lane * WCH
    lane_sub = lane * NSUBP

    def _load_chunk(r0, ch):
        c0 = ch * WCH
        copies = [
            pltpu.async_copy(scores.at[r0 + l, pl.ds(c0, WCH)],
                             chunk.at[pl.ds(l * WCH, WCH)], sem)
            for l in range(16)
        ]
        for cp in copies:
            cp.wait()

    for g in range(NGR):
        r0 = wid * RPW + g * 16

        @plsc.parallel_loop(0, NB, unroll=8)
        def _zb(b):
            hist[pl.ds(b * 16, 16)] = zeros

        copies = [
            pltpu.async_copy(submax.at[r0 + l, :],
                             chunk.at[pl.ds(l * NSUBP, NSUBP)], sem)
            for l in range(16)
        ]
        for cp in copies:
            cp.wait()

        @plsc.parallel_loop(0, NSUBP, unroll=8)
        def _p1_col(j):
            v = plsc.load_gather(chunk, [lane_sub + j])
            b = lax.bitcast_convert_type(v, jnp.int32)
            m = lax.shift_right_arithmetic(b, 31)
            u = b ^ (m | _I32MIN)
            t = lax.shift_right_logical(u, 21)
            plsc.addupdate_scatter(hist, [t * 16 + lane], ones)

        @plsc.parallel_loop(0, NB, unroll=8, carry=(zeros, zeros))
        def _th(b2, carry):
            cum, bstar = carry
            b = NB - 1 - b2
            h = hist[pl.ds(b * 16, 16)]
            newcum = cum + h
            crossed = (newcum >= TOPK) & (cum < TOPK)
            bstar = jnp.where(crossed, zeros + b, bstar)
            return (newcum, bstar)
        _, bstar = _th

        @plsc.parallel_loop(0, CAP + SB, unroll=8)
        def _zc(jj):
            ck[pl.ds(jj * 16, 16)] = zeros - 1

        def _p2_chunk(ch, ptr, _r0=r0):
            _load_chunk(_r0, ch)
            c0 = ch * WCH

            def _p2_sb(sb, ptr):
                j0 = sb * SB

                @plsc.parallel_loop(0, SB, unroll=8, carry=ptr)
                def _p2_col(jj, ptr):
                    j = j0 + jj
                    v = plsc.load_gather(chunk, [lane_chunk + j])
                    b = lax.bitcast_convert_type(v, jnp.int32)
                    m = lax.shift_right_arithmetic(b, 31)
                    u = b ^ (m | _I32MIN)
                    t = lax.shift_right_logical(u, 21)
                    keep = t >= bstar
                    kp = ~u
                    colv = zeros + (j + c0)
                    plsc.store_scatter(ck, [ptr * 16 + lane], kp, mask=keep)
                    plsc.store_scatter(ci, [ptr * 16 + lane], colv, mask=keep)
                    return ptr + jnp.where(keep, 1, 0)
                return jnp.minimum(_p2_col, CAP)
            return lax.fori_loop(0, WCH // SB, _p2_sb, ptr)
        ptr_fin = lax.fori_loop(0, NCH, _p2_chunk, zeros)
        cnt = jnp.max(ptr_fin) + 1
        cnt = jnp.minimum((cnt + 7) & ~7, CAP)

        pairs = [(ck, ci), (ck2, ci2)]
        cur = 0
        for p in range(4):
            src_k, src_i = pairs[cur]
            dst_k, dst_i = pairs[1 - cur]
            shift = 8 * p

            @plsc.parallel_loop(0, 256, unroll=8)
            def _zr(d):
                rhist[pl.ds(d * 16, 16)] = zeros

            def _bh_loop(_sk=src_k, _sh=shift):
                @plsc.parallel_loop(0, cnt, unroll=8)
                def _bh(j):
                    kk = _sk[pl.ds(j * 16, 16)]
                    d = lax.shift_right_logical(kk, _sh) & 255
                    plsc.addupdate_scatter(rhist, [d * 16 + lane], ones)
            _bh_loop()

            def _pf(d, acc):
                h = rhist[pl.ds(d * 16, 16)]
                rhist[pl.ds(d * 16, 16)] = acc
                return acc + h
            lax.fori_loop(0, 256, _pf, zeros)

            def _pm(j, c, _sk=src_k, _si=src_i, _dk=dst_k, _di=dst_i,
                    _sh=shift):
                kk = _sk[pl.ds(j * 16, 16)]
                ii = _si[pl.ds(j * 16, 16)]
                d = lax.shift_right_logical(kk, _sh) & 255
                o = plsc.load_gather(rhist, [d * 16 + lane])
                plsc.store_scatter(_dk, [o * 16 + lane], kk)
                plsc.store_scatter(_di, [o * 16 + lane], ii)
                plsc.addupdate_scatter(rhist, [d * 16 + lane], ones)
                return c
            lax.fori_loop(0, cnt, _pm, 0)
            cur = 1 - cur

        fin_k, fin_i = pairs[cur]

        def _ow(j, c):
            kk = fin_k[pl.ds(j * 16, 16)]
            u = ~kk
            neg = lax.shift_right_arithmetic(u, 31)
            mask32 = _I32MIN | (~neg & 0x7FFFFFFF)
            bfin = u ^ mask32
            v = lax.bitcast_convert_type(bfin, jnp.float32)
            plsc.store_scatter(outv, [lane * TOPK + j], v)
            plsc.store_scatter(outi, [lane * TOPK + j],
                               fin_i[pl.ds(j * 16, 16)])
            return c
        lax.fori_loop(0, TOPK, _ow, 0)
        pltpu.async_copy(outv, vals_out.at[pl.ds(r0 * TOPK, 16 * TOPK)],
                         sem).wait()
        pltpu.async_copy(outi, idx_out.at[pl.ds(r0 * TOPK, 16 * TOPK)],
                         sem).wait()


_topk_sc = functools.partial(
    pl.kernel,
    out_type=(jax.ShapeDtypeStruct((Q * TOPK,), jnp.float32),
              jax.ShapeDtypeStruct((Q * TOPK,), jnp.int32)),
    name="sc_topk",
    mesh=plsc.VectorSubcoreMesh(core_axis_name="c", subcore_axis_name="s"),
    compiler_params=pltpu.CompilerParams(needs_layout_passes=False),
    scratch_types=[
        pltpu.VMEM((16 * WCH,), jnp.float32),
        pltpu.VMEM((NB * 16,), jnp.int32),
        pltpu.VMEM(((CAP + SB) * 16,), jnp.int32),
        pltpu.VMEM(((CAP + SB) * 16,), jnp.int32),
        pltpu.VMEM((CAP * 16,), jnp.int32),
        pltpu.VMEM((CAP * 16,), jnp.int32),
        pltpu.VMEM((256 * 16,), jnp.int32),
        pltpu.VMEM((16 * TOPK,), jnp.float32),
        pltpu.VMEM((16 * TOPK,), jnp.int32),
        pltpu.SemaphoreType.DMA,
    ],
)(_topk_sc_body)


def kernel(question_hidden, W_pool, b_pool, passages, topk):
    b2 = b_pool.reshape(1, D)
    embeds = pl.pallas_call(
        _pooler_body,
        out_shape=jax.ShapeDtypeStruct((Q, D), jnp.float32),
    )(question_hidden, W_pool, b2)

    passages_p = jnp.pad(passages, ((0, KP - K), (0, 0)))
    scores, submax = pl.pallas_call(
        _scores_body,
        grid=(GRID,),
        in_specs=[
            pl.BlockSpec((Q, D), lambda i: (0, 0)),
            pl.BlockSpec((TK, D), lambda i: (i, 0)),
        ],
        out_specs=[
            pl.BlockSpec((Q, TK), lambda i: (0, i)),
            pl.BlockSpec((1, Q, TK // SUBCH), lambda i: (i, 0, 0)),
        ],
        out_shape=[
            jax.ShapeDtypeStruct((Q, KP), jnp.float32),
            jax.ShapeDtypeStruct((GRID, Q, TK // SUBCH), jnp.float32),
        ],
    )(embeds, passages_p)
    submax = submax.transpose(1, 0, 2).reshape(Q, NSUB)
    submax = jnp.pad(submax, ((0, 0), (0, NSUBP - NSUB)),
                     constant_values=jnp.finfo(jnp.float32).min)

    values, idx = _topk_sc(scores, submax)
    return values.reshape(Q, TOPK), idx.reshape(Q, TOPK)


# speedup vs baseline: 1.0547x; 1.0547x over previous
import functools

import jax
import jax.numpy as jnp
from jax import lax
from jax.experimental import pallas as pl
from jax.experimental.pallas import tpu as pltpu
from jax.experimental.pallas import tpu_sc as plsc

Q = 1024
D = 768
K = 100000
TOPK = 100
TK = 2048
GRID = (K + TK - 1) // TK
KP = GRID * TK

NB = 2048
CAP = 1024
WCH = 1024
NCH = KP // WCH
SB = 64
SUBCH = 128
NSUB = KP // SUBCH
NSUBP = 896
NC = 2
NS = 16
NW = NC * NS
RPW = Q // NW
NGR = RPW // 16

_I32MIN = -0x80000000


def _pooler_body(qh_ref, w_ref, b_ref, e_ref):
    acc = jax.lax.dot_general(
        qh_ref[...], w_ref[...], (((1,), (0,)), ((), ())),
        preferred_element_type=jnp.float32,
    )
    e_ref[...] = jnp.tanh(acc + b_ref[...])


def _scores_body(e_ref, p_ref, s_ref, sm_ref):
    i = pl.program_id(0)
    acc = jax.lax.dot_general(
        e_ref[...], p_ref[...], (((1,), (1,)), ((), ())),
        preferred_element_type=jnp.float32,
    )
    col = i * TK + jax.lax.broadcasted_iota(jnp.int32, (Q, TK), 1)
    masked = jnp.where(col < K, acc, jnp.finfo(jnp.float32).min)
    s_ref[...] = masked
    sm_ref[...] = jnp.max(masked.reshape(Q, TK // SUBCH, SUBCH),
                          axis=2)[None]


def _topk_sc_body(scores, submax, vals_out, idx_out,
                  chunk, hist, ck, ci, ck2, ci2, rhist, outv, outi, sem):
    wid = lax.axis_index("s") * NC + lax.axis_index("c")
    lane = lax.iota(jnp.int32, 16)
    ones = jnp.ones((16,), jnp.int32)
    zeros = jnp.zeros((16,), jnp.int32)
    lane_chunk = lane * WCH
    lane_sub = lane * NSUBP

    def _load_chunk(r0, ch):
        c0 = ch * WCH
        copies = [
            pltpu.async_copy(scores.at[r0 + l, pl.ds(c0, WCH)],
                             chunk.at[pl.ds(l * WCH, WCH)], sem)
            for l in range(16)
        ]
        for cp in copies:
            cp.wait()

    for g in range(NGR):
        r0 = wid * RPW + g * 16

        @plsc.parallel_loop(0, NB, unroll=8)
        def _zb(b):
            hist[pl.ds(b * 16, 16)] = zeros

        copies = [
            pltpu.async_copy(submax.at[r0 + l, :],
                             chunk.at[pl.ds(l * NSUBP, NSUBP)], sem)
            for l in range(16)
        ]
        for cp in copies:
            cp.wait()

        @plsc.parallel_loop(0, NSUBP, unroll=8)
        def _p1_col(j):
            v = plsc.load_gather(chunk, [lane_sub + j])
            b = lax.bitcast_convert_type(v, jnp.int32)
            m = lax.shift_right_arithmetic(b, 31)
            u = b ^ (m | _I32MIN)
            t = lax.shift_right_logical(u, 21)
            plsc.addupdate_scatter(hist, [t * 16 + lane], ones)

        @plsc.parallel_loop(0, NB, unroll=8, carry=(zeros, zeros))
        def _th(b2, carry):
            cum, bstar = carry
            b = NB - 1 - b2
            h = hist[pl.ds(b * 16, 16)]
            newcum = cum + h
            crossed = (newcum >= TOPK) & (cum < TOPK)
            bstar = jnp.where(crossed, zeros + b, bstar)
            return (newcum, bstar)
        _, bstar = _th

        @plsc.parallel_loop(0, CAP, unroll=8)
        def _zc(jj):
            ck[pl.ds(jj * 16, 16)] = zeros - 1

        def _p2_chunk(ch, ptr, _r0=r0):
            _load_chunk(_r0, ch)
            c0 = ch * WCH

            @plsc.parallel_loop(0, WCH, unroll=8, carry=ptr)
            def _p2_col(j, ptr):
                v = plsc.load_gather(chunk, [lane_chunk + j])
                b = lax.bitcast_convert_type(v, jnp.int32)
                m = lax.shift_right_arithmetic(b, 31)
                u = b ^ (m | _I32MIN)
                t = lax.shift_right_logical(u, 21)
                keep = t >= bstar
                kp = ~u
                colv = zeros + (j + c0)
                plsc.store_scatter(ck, [ptr * 16 + lane], kp, mask=keep)
                plsc.store_scatter(ci, [ptr * 16 + lane], colv, mask=keep)
                return jnp.minimum(ptr + jnp.where(keep, 1, 0), CAP - 1)
            return _p2_col
        ptr_fin = lax.fori_loop(0, NCH, _p2_chunk, zeros)
        cnt = jnp.max(ptr_fin) + 1
        cnt = jnp.minimum((cnt + 7) & ~7, CAP)

        pairs = [(ck, ci), (ck2, ci2)]
        cur = 0
        for p in range(4):
            src_k, src_i = pairs[cur]
            dst_k, dst_i = pairs[1 - cur]
            shift = 8 * p

            @plsc.parallel_loop(0, 256, unroll=8)
            def _zr(d):
                rhist[pl.ds(d * 16, 16)] = zeros

            def _bh_loop(_sk=src_k, _sh=shift):
                @plsc.parallel_loop(0, cnt, unroll=8)
                def _bh(j):
                    kk = _sk[pl.ds(j * 16, 16)]
                    d = lax.shift_right_logical(kk, _sh) & 255
                    plsc.addupdate_scatter(rhist, [d * 16 + lane], ones)
            _bh_loop()

            def _pf(d, acc):
                h = rhist[pl.ds(d * 16, 16)]
                rhist[pl.ds(d * 16, 16)] = acc
                return acc + h
            lax.fori_loop(0, 256, _pf, zeros)

            def _pm(j, c, _sk=src_k, _si=src_i, _dk=dst_k, _di=dst_i,
                    _sh=shift):
                kk = _sk[pl.ds(j * 16, 16)]
                ii = _si[pl.ds(j * 16, 16)]
                d = lax.shift_right_logical(kk, _sh) & 255
                o = plsc.load_gather(rhist, [d * 16 + lane])
                plsc.store_scatter(_dk, [o * 16 + lane], kk)
                plsc.store_scatter(_di, [o * 16 + lane], ii)
                plsc.addupdate_scatter(rhist, [d * 16 + lane], ones)
                return c
            lax.fori_loop(0, cnt, _pm, 0)
            cur = 1 - cur

        fin_k, fin_i = pairs[cur]

        def _ow(j, c):
            kk = fin_k[pl.ds(j * 16, 16)]
            u = ~kk
            neg = lax.shift_right_arithmetic(u, 31)
            mask32 = _I32MIN | (~neg & 0x7FFFFFFF)
            bfin = u ^ mask32
            v = lax.bitcast_convert_type(bfin, jnp.float32)
            plsc.store_scatter(outv, [lane * TOPK + j], v)
            plsc.store_scatter(outi, [lane * TOPK + j],
                               fin_i[pl.ds(j * 16, 16)])
            return c
        lax.fori_loop(0, TOPK, _ow, 0)
        pltpu.async_copy(outv, vals_out.at[pl.ds(r0 * TOPK, 16 * TOPK)],
                         sem).wait()
        pltpu.async_copy(outi, idx_out.at[pl.ds(r0 * TOPK, 16 * TOPK)],
                         sem).wait()


_topk_sc = functools.partial(
    pl.kernel,
    out_type=(jax.ShapeDtypeStruct((Q * TOPK,), jnp.float32),
              jax.ShapeDtypeStruct((Q * TOPK,), jnp.int32)),
    name="sc_topk",
    mesh=plsc.VectorSubcoreMesh(core_axis_name="c", subcore_axis_name="s"),
    compiler_params=pltpu.CompilerParams(needs_layout_passes=False),
    scratch_types=[
        pltpu.VMEM((16 * WCH,), jnp.float32),
        pltpu.VMEM((NB * 16,), jnp.int32),
        pltpu.VMEM((CAP * 16,), jnp.int32),
        pltpu.VMEM((CAP * 16,), jnp.int32),
        pltpu.VMEM((CAP * 16,), jnp.int32),
        pltpu.VMEM((CAP * 16,), jnp.int32),
        pltpu.VMEM((256 * 16,), jnp.int32),
        pltpu.VMEM((16 * TOPK,), jnp.float32),
        pltpu.VMEM((16 * TOPK,), jnp.int32),
        pltpu.SemaphoreType.DMA,
    ],
)(_topk_sc_body)


def kernel(question_hidden, W_pool, b_pool, passages, topk):
    b2 = b_pool.reshape(1, D)
    embeds = pl.pallas_call(
        _pooler_body,
        out_shape=jax.ShapeDtypeStruct((Q, D), jnp.float32),
    )(question_hidden, W_pool, b2)

    passages_p = jnp.pad(passages, ((0, KP - K), (0, 0)))
    scores, submax = pl.pallas_call(
        _scores_body,
        grid=(GRID,),
        in_specs=[
            pl.BlockSpec((Q, D), lambda i: (0, 0)),
            pl.BlockSpec((TK, D), lambda i: (i, 0)),
        ],
        out_specs=[
            pl.BlockSpec((Q, TK), lambda i: (0, i)),
            pl.BlockSpec((1, Q, TK // SUBCH), lambda i: (i, 0, 0)),
        ],
        out_shape=[
            jax.ShapeDtypeStruct((Q, KP), jnp.float32),
            jax.ShapeDtypeStruct((GRID, Q, TK // SUBCH), jnp.float32),
        ],
    )(embeds, passages_p)
    submax = submax.transpose(1, 0, 2).reshape(Q, NSUB)
    submax = jnp.pad(submax, ((0, 0), (0, NSUBP - NSUB)),
                     constant_values=jnp.finfo(jnp.float32).min)

    values, idx = _topk_sc(scores, submax)
    return values.reshape(Q, TOPK), idx.reshape(Q, TOPK)


# speedup vs baseline: 1.1553x; 1.0954x over previous
import functools

import jax
import jax.numpy as jnp
from jax import lax
from jax.experimental import pallas as pl
from jax.experimental.pallas import tpu as pltpu
from jax.experimental.pallas import tpu_sc as plsc

Q = 1024
D = 768
K = 100000
TOPK = 100
TK = 2048
GRID = (K + TK - 1) // TK
KP = GRID * TK

NB = 2048
CAP = 1024
WCH = 512
NCH = KP // WCH
SB = 64
SUBCH = 128
NSUB = KP // SUBCH
NSUBP = 896
NC = 2
NS = 16
NW = NC * NS
RPW = Q // NW
NGR = RPW // 16

_I32MIN = -0x80000000


def _pooler_body(qh_ref, w_ref, b_ref, e_ref):
    acc = jax.lax.dot_general(
        qh_ref[...], w_ref[...], (((1,), (0,)), ((), ())),
        preferred_element_type=jnp.float32,
    )
    e_ref[...] = jnp.tanh(acc + b_ref[...])


def _scores_body(e_ref, p_ref, s_ref, sm_ref):
    i = pl.program_id(0)
    acc = jax.lax.dot_general(
        e_ref[...], p_ref[...], (((1,), (1,)), ((), ())),
        preferred_element_type=jnp.float32,
    )
    col = i * TK + jax.lax.broadcasted_iota(jnp.int32, (Q, TK), 1)
    masked = jnp.where(col < K, acc, jnp.finfo(jnp.float32).min)
    s_ref[...] = masked
    sm_ref[...] = jnp.max(masked.reshape(Q, TK // SUBCH, SUBCH),
                          axis=2)[None]


def _topk_sc_body(scores, submax, vals_out, idx_out,
                  chunkA, chunkB, hist, ck, ci, ck2, ci2, rhist, outv, outi,
                  semA, semB, semo):
    wid = lax.axis_index("s") * NC + lax.axis_index("c")
    lane = lax.iota(jnp.int32, 16)
    ones = jnp.ones((16,), jnp.int32)
    zeros = jnp.zeros((16,), jnp.int32)
    lane_chunk = lane * WCH
    lane_sub = lane * NSUBP

    def _issue(r0, ch, buf, s):
        c0 = ch * WCH
        for l in range(16):
            pltpu.async_copy(scores.at[r0 + l, pl.ds(c0, WCH)],
                             buf.at[pl.ds(l * WCH, WCH)], s)

    def _drain(r0, buf, s):
        for l in range(16):
            pltpu.make_async_copy(scores.at[r0 + l, pl.ds(0, WCH)],
                                  buf.at[pl.ds(l * WCH, WCH)], s).wait()

    for g in range(NGR):
        r0 = wid * RPW + g * 16

        @plsc.parallel_loop(0, NB, unroll=8)
        def _zb(b):
            hist[pl.ds(b * 16, 16)] = zeros

        copies = [
            pltpu.async_copy(submax.at[r0 + l, :],
                             ck2.at[pl.ds(l * NSUBP, NSUBP)], semA)
            for l in range(16)
        ]
        for cp in copies:
            cp.wait()

        @plsc.parallel_loop(0, NSUBP, unroll=8)
        def _p1_col(j):
            b = plsc.load_gather(ck2, [lane_sub + j])
            m = lax.shift_right_arithmetic(b, 31)
            u = b ^ (m | _I32MIN)
            t = lax.shift_right_logical(u, 21)
            plsc.addupdate_scatter(hist, [t * 16 + lane], ones)

        @plsc.parallel_loop(0, NB, unroll=8, carry=(zeros, zeros))
        def _th(b2, carry):
            cum, bstar = carry
            b = NB - 1 - b2
            h = hist[pl.ds(b * 16, 16)]
            newcum = cum + h
            crossed = (newcum >= TOPK) & (cum < TOPK)
            bstar = jnp.where(crossed, zeros + b, bstar)
            return (newcum, bstar)
        _, bstar = _th

        @plsc.parallel_loop(0, CAP, unroll=8)
        def _zc(jj):
            ck[pl.ds(jj * 16, 16)] = zeros - 1

        def _scan_buf(buf, c0, ptr):
            @plsc.parallel_loop(0, WCH, unroll=8, carry=ptr)
            def _p2_col(j, ptr):
                v = plsc.load_gather(buf, [lane_chunk + j])
                b = lax.bitcast_convert_type(v, jnp.int32)
                m = lax.shift_right_arithmetic(b, 31)
                u = b ^ (m | _I32MIN)
                t = lax.shift_right_logical(u, 21)
                keep = t >= bstar
                kp = ~u
                colv = zeros + (j + c0)
                plsc.store_scatter(ck, [ptr * 16 + lane], kp, mask=keep)
                plsc.store_scatter(ci, [ptr * 16 + lane], colv, mask=keep)
                return jnp.minimum(ptr + jnp.where(keep, 1, 0), CAP - 1)
            return _p2_col

        _issue(r0, 0, chunkA, semA)

        def _p2_pair(p, ptr, _r0=r0):
            chA = 2 * p
            _issue(_r0, chA + 1, chunkB, semB)
            _drain(_r0, chunkA, semA)
            ptr = _scan_buf(chunkA, chA * WCH, ptr)

            @pl.when(chA + 2 < NCH)
            def _():
                _issue(_r0, chA + 2, chunkA, semA)
            _drain(_r0, chunkB, semB)
            ptr = _scan_buf(chunkB, (chA + 1) * WCH, ptr)
            return ptr
        ptr_fin = lax.fori_loop(0, NCH // 2, _p2_pair, zeros)
        cnt = jnp.max(ptr_fin) + 1
        cnt = jnp.minimum((cnt + 7) & ~7, CAP)

        pairs = [(ck, ci), (ck2, ci2)]
        cur = 0
        for p in range(4):
            src_k, src_i = pairs[cur]
            dst_k, dst_i = pairs[1 - cur]
            shift = 8 * p

            @plsc.parallel_loop(0, 256, unroll=8)
            def _zr(d):
                rhist[pl.ds(d * 16, 16)] = zeros

            def _bh_loop(_sk=src_k, _sh=shift):
                @plsc.parallel_loop(0, cnt, unroll=8)
                def _bh(j):
                    kk = _sk[pl.ds(j * 16, 16)]
                    d = lax.shift_right_logical(kk, _sh) & 255
                    plsc.addupdate_scatter(rhist, [d * 16 + lane], ones)
            _bh_loop()

            def _pf(d, acc):
                h = rhist[pl.ds(d * 16, 16)]
                rhist[pl.ds(d * 16, 16)] = acc
                return acc + h
            lax.fori_loop(0, 256, _pf, zeros)

            def _pm(j, c, _sk=src_k, _si=src_i, _dk=dst_k, _di=dst_i,
                    _sh=shift):
                kk = _sk[pl.ds(j * 16, 16)]
                ii = _si[pl.ds(j * 16, 16)]
                d = lax.shift_right_logical(kk, _sh) & 255
                o = plsc.load_gather(rhist, [d * 16 + lane])
                plsc.store_scatter(_dk, [o * 16 + lane], kk)
                plsc.store_scatter(_di, [o * 16 + lane], ii)
                plsc.addupdate_scatter(rhist, [d * 16 + lane], ones)
                return c
            lax.fori_loop(0, cnt, _pm, 0)
            cur = 1 - cur

        fin_k, fin_i = pairs[cur]

        def _ow(j, c):
            kk = fin_k[pl.ds(j * 16, 16)]
            u = ~kk
            neg = lax.shift_right_arithmetic(u, 31)
            mask32 = _I32MIN | (~neg & 0x7FFFFFFF)
            bfin = u ^ mask32
            v = lax.bitcast_convert_type(bfin, jnp.float32)
            plsc.store_scatter(outv, [lane * TOPK + j], v)
            plsc.store_scatter(outi, [lane * TOPK + j],
                               fin_i[pl.ds(j * 16, 16)])
            return c
        lax.fori_loop(0, TOPK, _ow, 0)
        pltpu.async_copy(outv, vals_out.at[pl.ds(r0 * TOPK, 16 * TOPK)],
                         semo).wait()
        pltpu.async_copy(outi, idx_out.at[pl.ds(r0 * TOPK, 16 * TOPK)],
                         semo).wait()


_topk_sc = functools.partial(
    pl.kernel,
    out_type=(jax.ShapeDtypeStruct((Q * TOPK,), jnp.float32),
              jax.ShapeDtypeStruct((Q * TOPK,), jnp.int32)),
    name="sc_topk",
    mesh=plsc.VectorSubcoreMesh(core_axis_name="c", subcore_axis_name="s"),
    compiler_params=pltpu.CompilerParams(needs_layout_passes=False),
    scratch_types=[
        pltpu.VMEM((16 * WCH,), jnp.float32),
        pltpu.VMEM((16 * WCH,), jnp.float32),
        pltpu.VMEM((NB * 16,), jnp.int32),
        pltpu.VMEM((CAP * 16,), jnp.int32),
        pltpu.VMEM((CAP * 16,), jnp.int32),
        pltpu.VMEM((CAP * 16,), jnp.int32),
        pltpu.VMEM((CAP * 16,), jnp.int32),
        pltpu.VMEM((256 * 16,), jnp.int32),
        pltpu.VMEM((16 * TOPK,), jnp.float32),
        pltpu.VMEM((16 * TOPK,), jnp.int32),
        pltpu.SemaphoreType.DMA,
        pltpu.SemaphoreType.DMA,
        pltpu.SemaphoreType.DMA,
    ],
)(_topk_sc_body)


def kernel(question_hidden, W_pool, b_pool, passages, topk):
    b2 = b_pool.reshape(1, D)
    embeds = pl.pallas_call(
        _pooler_body,
        out_shape=jax.ShapeDtypeStruct((Q, D), jnp.float32),
    )(question_hidden, W_pool, b2)

    passages_p = jnp.pad(passages, ((0, KP - K), (0, 0)))
    scores, submax = pl.pallas_call(
        _scores_body,
        grid=(GRID,),
        in_specs=[
            pl.BlockSpec((Q, D), lambda i: (0, 0)),
            pl.BlockSpec((TK, D), lambda i: (i, 0)),
        ],
        out_specs=[
            pl.BlockSpec((Q, TK), lambda i: (0, i)),
            pl.BlockSpec((1, Q, TK // SUBCH), lambda i: (i, 0, 0)),
        ],
        out_shape=[
            jax.ShapeDtypeStruct((Q, KP), jnp.float32),
            jax.ShapeDtypeStruct((GRID, Q, TK // SUBCH), jnp.float32),
        ],
    )(embeds, passages_p)
    submax = submax.transpose(1, 0, 2).reshape(Q, NSUB)
    submax = jnp.pad(submax, ((0, 0), (0, NSUBP - NSUB)),
                     constant_values=jnp.finfo(jnp.float32).min)
    submax = lax.bitcast_convert_type(submax, jnp.int32)

    values, idx = _topk_sc(scores, submax)
    return values.reshape(Q, TOPK), idx.reshape(Q, TOPK)


# speedup vs baseline: 1.1555x; 1.0002x over previous
import functools

import jax
import jax.numpy as jnp
from jax import lax
from jax.experimental import pallas as pl
from jax.experimental.pallas import tpu as pltpu
from jax.experimental.pallas import tpu_sc as plsc

Q = 1024
D = 768
K = 100000
TOPK = 100
TK = 2048
GRID = (K + TK - 1) // TK
KP = GRID * TK

NB = 2048
CAP = 1024
WCH = 512
NCH = KP // WCH
SUBCH = 128
NSUB = KP // SUBCH
NSUBP = 896
NC = 2
NS = 16
NW = NC * NS
RPW = Q // NW
NGR = RPW // 16

_I32MIN = -0x80000000


def _pooler_body(qh_ref, w_ref, b_ref, e_ref):
    acc = jax.lax.dot_general(
        qh_ref[...], w_ref[...], (((1,), (0,)), ((), ())),
        preferred_element_type=jnp.float32,
    )
    e_ref[...] = jnp.tanh(acc + b_ref[...])


def _scores_body(e_ref, p_ref, s_ref, sm_ref):
    i = pl.program_id(0)
    acc = jax.lax.dot_general(
        e_ref[...], p_ref[...], (((1,), (1,)), ((), ())),
        preferred_element_type=jnp.float32,
    )
    col = i * TK + jax.lax.broadcasted_iota(jnp.int32, (Q, TK), 1)
    masked = jnp.where(col < K, acc, jnp.finfo(jnp.float32).min)
    s_ref[...] = masked
    sm_ref[...] = jnp.max(masked.reshape(Q, TK // SUBCH, SUBCH),
                          axis=2)[None]


def _topk_sc_body(scores, submax, vals_out, idx_out,
                  chunkA, chunkB, hist, ck, ci, ck2, ci2, rhist, outv, outi,
                  semA, semB, semo):
    wid = lax.axis_index("s") * NC + lax.axis_index("c")
    lane = lax.iota(jnp.int32, 16)
    ones = jnp.ones((16,), jnp.int32)
    zeros = jnp.zeros((16,), jnp.int32)
    lane_chunk = lane * WCH
    lane_sub = lane * NSUBP

    def _issue(r0, ch, buf, s):
        c0 = ch * WCH
        for l in range(16):
            pltpu.async_copy(scores.at[r0 + l, pl.ds(c0, WCH)],
                             buf.at[pl.ds(l * WCH, WCH)], s)

    def _drain(r0, buf, s):
        for l in range(16):
            pltpu.make_async_copy(scores.at[r0 + l, pl.ds(0, WCH)],
                                  buf.at[pl.ds(l * WCH, WCH)], s).wait()

    for g in range(NGR):
        r0 = wid * RPW + g * 16

        @plsc.parallel_loop(0, NB, unroll=8)
        def _zb(b):
            hist[pl.ds(b * 16, 16)] = zeros

        copies = [
            pltpu.async_copy(submax.at[r0 + l, :],
                             ck2.at[pl.ds(l * NSUBP, NSUBP)], semA)
            for l in range(16)
        ]
        for cp in copies:
            cp.wait()

        @plsc.parallel_loop(0, NSUBP, unroll=8)
        def _p1_col(j):
            b = plsc.load_gather(ck2, [lane_sub + j])
            m = lax.shift_right_arithmetic(b, 31)
            u = b ^ (m | _I32MIN)
            t = lax.shift_right_logical(u, 21)
            plsc.addupdate_scatter(hist, [t * 16 + lane], ones)

        @plsc.parallel_loop(0, NB, unroll=8, carry=(zeros, zeros))
        def _th(b2, carry):
            cum, bstar = carry
            b = NB - 1 - b2
            h = hist[pl.ds(b * 16, 16)]
            newcum = cum + h
            crossed = (newcum >= TOPK) & (cum < TOPK)
            bstar = jnp.where(crossed, zeros + b, bstar)
            return (newcum, bstar)
        _, bstar = _th

        @plsc.parallel_loop(0, CAP, unroll=8)
        def _zc(jj):
            ck[pl.ds(jj * 16, 16)] = zeros - 1

        def _scan_buf(buf, c0, ptr):
            @plsc.parallel_loop(0, WCH, unroll=8, carry=ptr)
            def _p2_col(j, ptr):
                v = plsc.load_gather(buf, [lane_chunk + j])
                b = lax.bitcast_convert_type(v, jnp.int32)
                m = lax.shift_right_arithmetic(b, 31)
                u = b ^ (m | _I32MIN)
                t = lax.shift_right_logical(u, 21)
                keep = t >= bstar
                kp = ~u
                colv = zeros + (j + c0)
                plsc.store_scatter(ck, [ptr * 16 + lane], kp, mask=keep)
                plsc.store_scatter(ci, [ptr * 16 + lane], colv, mask=keep)
                return jnp.minimum(ptr + jnp.where(keep, 1, 0), CAP - 1)
            return _p2_col

        _issue(r0, 0, chunkA, semA)

        def _p2_pair(p, ptr, _r0=r0):
            chA = 2 * p
            _issue(_r0, chA + 1, chunkB, semB)
            _drain(_r0, chunkA, semA)
            ptr = _scan_buf(chunkA, chA * WCH, ptr)

            @pl.when(chA + 2 < NCH)
            def _():
                _issue(_r0, chA + 2, chunkA, semA)
            _drain(_r0, chunkB, semB)
            ptr = _scan_buf(chunkB, (chA + 1) * WCH, ptr)
            return ptr
        ptr_fin = lax.fori_loop(0, NCH // 2, _p2_pair, zeros)
        cnt = jnp.max(ptr_fin) + 1
        cnt = jnp.minimum((cnt + 7) & ~7, CAP)

        pairs = [(ck, ci), (ck2, ci2)]
        cur = 0
        for p in range(4):
            src_k, src_i = pairs[cur]
            dst_k, dst_i = pairs[1 - cur]
            shift = 8 * p

            @plsc.parallel_loop(0, 256, unroll=8)
            def _zr(d):
                rhist[pl.ds(d * 16, 16)] = zeros

            def _bh_loop(_sk=src_k, _sh=shift):
                @plsc.parallel_loop(0, cnt, unroll=8)
                def _bh(j):
                    kk = _sk[pl.ds(j * 16, 16)]
                    d = lax.shift_right_logical(kk, _sh) & 255
                    plsc.addupdate_scatter(rhist, [d * 16 + lane], ones)
            _bh_loop()

            def _pf(d, acc):
                h = rhist[pl.ds(d * 16, 16)]
                rhist[pl.ds(d * 16, 16)] = acc
                return acc + h
            lax.fori_loop(0, 256, _pf, zeros)

            def _pm(j, c, _sk=src_k, _si=src_i, _dk=dst_k, _di=dst_i,
                    _sh=shift):
                kk = _sk[pl.ds(j * 16, 16)]
                ii = _si[pl.ds(j * 16, 16)]
                d = lax.shift_right_logical(kk, _sh) & 255
                o = plsc.load_gather(rhist, [d * 16 + lane])
                plsc.store_scatter(_dk, [o * 16 + lane], kk)
                plsc.store_scatter(_di, [o * 16 + lane], ii)
                plsc.addupdate_scatter(rhist, [d * 16 + lane], ones)
                return c
            lax.fori_loop(0, cnt, _pm, 0)
            cur = 1 - cur

        fin_k, fin_i = pairs[cur]

        def _ow(j, c):
            kk = fin_k[pl.ds(j * 16, 16)]
            u = ~kk
            neg = lax.shift_right_arithmetic(u, 31)
            mask32 = _I32MIN | (~neg & 0x7FFFFFFF)
            bfin = u ^ mask32
            v = lax.bitcast_convert_type(bfin, jnp.float32)
            plsc.store_scatter(outv, [lane * TOPK + j], v)
            plsc.store_scatter(outi, [lane * TOPK + j],
                               fin_i[pl.ds(j * 16, 16)])
            return c
        lax.fori_loop(0, TOPK, _ow, 0)
        pltpu.async_copy(outv, vals_out.at[pl.ds(r0 * TOPK, 16 * TOPK)],
                         semo).wait()
        pltpu.async_copy(outi, idx_out.at[pl.ds(r0 * TOPK, 16 * TOPK)],
                         semo).wait()


_topk_sc = functools.partial(
    pl.kernel,
    out_type=(jax.ShapeDtypeStruct((Q * TOPK,), jnp.float32),
              jax.ShapeDtypeStruct((Q * TOPK,), jnp.int32)),
    name="sc_topk",
    mesh=plsc.VectorSubcoreMesh(core_axis_name="c", subcore_axis_name="s"),
    compiler_params=pltpu.CompilerParams(needs_layout_passes=False),
    scratch_types=[
        pltpu.VMEM((16 * WCH,), jnp.float32),
        pltpu.VMEM((16 * WCH,), jnp.float32),
        pltpu.VMEM((NB * 16,), jnp.int32),
        pltpu.VMEM((CAP * 16,), jnp.int32),
        pltpu.VMEM((CAP * 16,), jnp.int32),
        pltpu.VMEM((CAP * 16,), jnp.int32),
        pltpu.VMEM((CAP * 16,), jnp.int32),
        pltpu.VMEM((256 * 16,), jnp.int32),
        pltpu.VMEM((16 * TOPK,), jnp.float32),
        pltpu.VMEM((16 * TOPK,), jnp.int32),
        pltpu.SemaphoreType.DMA,
        pltpu.SemaphoreType.DMA,
        pltpu.SemaphoreType.DMA,
    ],
)(_topk_sc_body)


def kernel(question_hidden, W_pool, b_pool, passages, topk):
    b2 = b_pool.reshape(1, D)
    embeds = pl.pallas_call(
        _pooler_body,
        out_shape=jax.ShapeDtypeStruct((Q, D), jnp.float32),
    )(question_hidden, W_pool, b2)

    passages_p = jnp.pad(passages, ((0, KP - K), (0, 0)))
    scores, submax = pl.pallas_call(
        _scores_body,
        grid=(GRID,),
        in_specs=[
            pl.BlockSpec((Q, D), lambda i: (0, 0)),
            pl.BlockSpec((TK, D), lambda i: (i, 0)),
        ],
        out_specs=[
            pl.BlockSpec((Q, TK), lambda i: (0, i)),
            pl.BlockSpec((1, Q, TK // SUBCH), lambda i: (i, 0, 0)),
        ],
        out_shape=[
            jax.ShapeDtypeStruct((Q, KP), jnp.float32),
            jax.ShapeDtypeStruct((GRID, Q, TK // SUBCH), jnp.float32),
        ],
    )(embeds, passages_p)
    submax = submax.transpose(1, 0, 2).reshape(Q, NSUB)
    submax = jnp.pad(submax, ((0, 0), (0, NSUBP - NSUB)),
                     constant_values=jnp.finfo(jnp.float32).min)
    submax = lax.bitcast_convert_type(submax, jnp.int32)

    values, idx = _topk_sc(scores, submax)
    return values.reshape(Q, TOPK), idx.reshape(Q, TOPK)
